# Initial kernel scaffold; baseline (speedup 1.0000x reference)
import jax
import jax.numpy as jnp
from jax.experimental import pallas as pl


def kernel(x, edge_index, fake_x, treat_idx, control_idx, W1, att_src1, att_dst1, b1, W2, att_src2, att_dst2, b2, Wy1, by1, Wy0, by0):
    raise NotImplementedError("write your pallas kernel here")


# speedup vs baseline: 1.4507x; 1.4507x over previous
import jax
import jax.numpy as jnp
from jax.experimental import pallas as pl
from jax.experimental.pallas import tpu as pltpu

N = 10000
E = 320000
D = 128
H = 128


def _mm_kernel(x_ref, w_ref, o_ref):
    o_ref[...] = jnp.dot(x_ref[...], w_ref[...], preferred_element_type=jnp.float32)


def _matmul(x, w):
    m, k = x.shape
    _, n = w.shape
    bm = 1000
    assert m % bm == 0
    grid = (m // bm,)
    return pl.pallas_call(
        _mm_kernel,
        grid=grid,
        in_specs=[
            pl.BlockSpec((bm, k), lambda i: (i, 0)),
            pl.BlockSpec((k, n), lambda i: (0, 0)),
        ],
        out_specs=pl.BlockSpec((bm, n), lambda i: (i, 0)),
        out_shape=jax.ShapeDtypeStruct((m, n), jnp.float32),
    )(x, w)


def _gat_edge_phase(h, a_src, a_dst, src, dst, nseg):
    e = jax.nn.leaky_relu(a_src[src] + a_dst[dst], negative_slope=0.2)
    ex = jnp.exp(e)
    den = jax.ops.segment_sum(ex, dst, num_segments=nseg)
    alpha = ex / (den[dst] + 1e-16)
    return jax.ops.segment_sum(h[src] * alpha[:, None], dst, num_segments=nseg)


def kernel(x, edge_index, fake_x, treat_idx, control_idx,
           W1, att_src1, att_dst1, b1,
           W2, att_src2, att_dst2, b2,
           Wy1, by1, Wy0, by0):
    loop = jnp.arange(N, dtype=edge_index.dtype)
    src = jnp.concatenate([edge_index[0], loop])
    dst = jnp.concatenate([edge_index[1], loop])

    xb = jnp.concatenate([x, fake_x], axis=0)
    h1 = _matmul(xb, W1)
    a_src1 = h1 @ att_src1
    a_dst1 = h1 @ att_dst1
    o1x = _gat_edge_phase(h1[:N], a_src1[:N], a_dst1[:N], src, dst, N)
    o1f = _gat_edge_phase(h1[N:], a_src1[N:], a_dst1[N:], src, dst, N)
    z1 = jax.nn.relu(jnp.concatenate([o1x, o1f], axis=0) + b1)
    h2 = _matmul(z1, W2)
    a_src2 = h2 @ att_src2
    a_dst2 = h2 @ att_dst2
    o2x = _gat_edge_phase(h2[:N], a_src2[:N], a_dst2[:N], src, dst, N)
    o2f = _gat_edge_phase(h2[N:], a_src2[N:], a_dst2[N:], src, dst, N)
    xZ2 = o2x + b2
    xfZ2 = o2f + b2

    z1v = jax.nn.leaky_relu(xZ2 @ Wy1 + by1, negative_slope=0.01).squeeze(-1)
    z0v = jax.nn.leaky_relu(xZ2 @ Wy0 + by0, negative_slope=0.01).squeeze(-1)
    zf1v = jax.nn.leaky_relu(xfZ2 @ Wy1 + by1, negative_slope=0.01).squeeze(-1)
    zf0v = jax.nn.leaky_relu(xfZ2 @ Wy0 + by0, negative_slope=0.01).squeeze(-1)
    y1 = z1v[treat_idx]
    yc0 = zf0v[treat_idx]
    y0 = z0v[control_idx]
    yc1 = zf1v[control_idx]
    return (y1, yc0, y0, yc1, xZ2, xfZ2)


# speedup vs baseline: 11.7504x; 8.0996x over previous
import functools

import jax
import jax.numpy as jnp
from jax import lax
from jax.experimental import pallas as pl
from jax.experimental.pallas import tpu as pltpu
from jax.experimental.pallas import tpu_sc as plsc

N = 10000
NP = 10240
E = 320000
E2 = E + N
E2P = 330240
PADE = E2P - E2
NT = 16
EPT = E2P // NT
CHUNK = 4128
NCA = EPT // CHUNK
NCB = E2P // CHUNK
DR = NP // 128
CPT = 4
NCOLP = 2
TSEL = 5000
TP = 5120

_mesh = plsc.VectorSubcoreMesh(core_axis_name="c", subcore_axis_name="s")
_scparams = pltpu.CompilerParams(needs_layout_passes=False)


def _mmT_body(relu_in):
    def body(x_ref, w_ref, a_ref, o_ref):
        xb = x_ref[0]
        if relu_in:
            xb = jnp.maximum(xb, 0.0)
        hb = lax.dot_general(
            w_ref[...], xb, (((0,), (0,)), ((), ())),
            preferred_element_type=jnp.float32)
        ab = lax.dot_general(
            a_ref[...], hb, (((0,), (0,)), ((), ())),
            precision=lax.Precision.HIGHEST,
            preferred_element_type=jnp.float32)
        o_ref[0, :128, :] = hb
        o_ref[0, 128:130, :] = ab
    return body


def _dense(xt, w, attmat, relu_in):
    BN = 2048
    return pl.pallas_call(
        _mmT_body(relu_in),
        grid=(2, NP // BN),
        in_specs=[
            pl.BlockSpec((1, 128, BN), lambda c, j: (c, 0, j)),
            pl.BlockSpec((128, 128), lambda c, j: (0, 0)),
            pl.BlockSpec((128, 2), lambda c, j: (0, 0)),
        ],
        out_specs=pl.BlockSpec((1, 130, BN), lambda c, j: (c, 0, j)),
        out_shape=jax.ShapeDtypeStruct((2, 130, NP), jnp.float32),
    )(xt, w, attmat)


def _heads_body(x_ref, w_ref, b_ref, o_ref):
    z = lax.dot_general(w_ref[...], x_ref[0], (((0,), (0,)), ((), ())),
                        precision=lax.Precision.HIGHEST,
                        preferred_element_type=jnp.float32)
    z = z + b_ref[...]
    o_ref[0] = jnp.where(z < 0, z * jnp.float32(0.01), z)


def _heads(out2, wy, by):
    BN = 2048
    return pl.pallas_call(
        _heads_body,
        grid=(2, NP // BN),
        in_specs=[
            pl.BlockSpec((1, 128, BN), lambda c, j: (c, 0, j)),
            pl.BlockSpec((128, 2), lambda c, j: (0, 0)),
            pl.BlockSpec((2, 1), lambda c, j: (0, 0)),
        ],
        out_specs=pl.BlockSpec((1, 2, BN), lambda c, j: (c, 0, j)),
        out_shape=jax.ShapeDtypeStruct((2, 2, NP), jnp.float32),
    )(out2, wy, by)


@functools.partial(
    pl.kernel,
    out_type=[jax.ShapeDtypeStruct((2 * 128 * NP,), jnp.float32),
              jax.ShapeDtypeStruct((2 * E2P,), jnp.float32)],
    mesh=_mesh,
    compiler_params=_scparams,
    scratch_types=[
        pltpu.VMEM((2 * NP,), jnp.float32),
        pltpu.VMEM((DR, 128), jnp.float32),
        pltpu.VMEM((CPT * NP,), jnp.float32),
        pltpu.VMEM((CPT * NP,), jnp.float32),
        pltpu.VMEM((CHUNK,), jnp.int32),
        pltpu.VMEM((CHUNK,), jnp.int32),
        pltpu.VMEM((CHUNK,), jnp.float32),
        pltpu.VMEM((128,), jnp.float32),
        pltpu.VMEM((DR,), jnp.int32),
        pltpu.VMEM_SHARED((DR, 128), jnp.float32),
    ],
)
def _edge_kernel(hcat_hbm, src_hbm, dst_hbm, bias_hbm, out_hbm, alpha_hbm,
                 ab, den, hs, os_, sb, db, eb, biasb, io,
                 den_sh):
    cid = lax.axis_index("c")
    sid = lax.axis_index("s")
    hoff = cid * (130 * NP)
    aoff = cid * E2P
    ebase = sid * EPT
    zero = jnp.zeros((16,), jnp.float32)

    for j in range(DR):
        for jj in range(8):
            den[j, pl.ds(jj * 16, 16)] = zero
    for j in range(DR // 16):
        io[pl.ds(j * 16, 16)] = lax.iota(jnp.int32, 16) + j * 16

    @pl.when(sid == 0)
    def _():
        pltpu.sync_copy(den, den_sh)

    pltpu.sync_copy(hcat_hbm.at[pl.ds(hoff + 128 * NP, 2 * NP)], ab)
    pltpu.sync_copy(bias_hbm, biasb)
    plsc.subcore_barrier()

    for j in range(NCA):
        base = ebase + j * CHUNK
        pltpu.sync_copy(src_hbm.at[pl.ds(base, CHUNK)], sb)
        pltpu.sync_copy(dst_hbm.at[pl.ds(base, CHUNK)], db)

        def body_a(i, _):
            s16 = sb[pl.ds(i * 16, 16)]
            d16 = db[pl.ds(i * 16, 16)]
            a = plsc.load_gather(ab, [s16]) + plsc.load_gather(ab, [d16 + NP])
            e = jnp.where(a < 0, a * jnp.float32(0.2), a)
            ex = jnp.exp(e)
            plsc.addupdate_scatter(den, [d16 >> 7, d16 & 127], ex)
            eb[pl.ds(i * 16, 16)] = ex
            return 0

        lax.fori_loop(0, CHUNK // 16, body_a, 0)
        pltpu.sync_copy(eb, alpha_hbm.at[pl.ds(aoff + base, CHUNK)])

    pltpu.sync_copy(den, den_sh.at[io], add=True)
    plsc.subcore_barrier()
    pltpu.sync_copy(den_sh, den)

    for j in range(NCA):
        base = ebase + j * CHUNK
        pltpu.sync_copy(dst_hbm.at[pl.ds(base, CHUNK)], db)
        pltpu.sync_copy(alpha_hbm.at[pl.ds(aoff + base, CHUNK)], eb)

        def body_a3(i, _):
            d16 = db[pl.ds(i * 16, 16)]
            ex = eb[pl.ds(i * 16, 16)]
            dd = plsc.load_gather(den, [d16 >> 7, d16 & 127])
            eb[pl.ds(i * 16, 16)] = ex / dd
            return 0

        lax.fori_loop(0, CHUNK // 16, body_a3, 0)
        pltpu.sync_copy(eb, alpha_hbm.at[pl.ds(aoff + base, CHUNK)])
    plsc.subcore_barrier()

    for p in range(NCOLP):
        colbase = p * 64 + sid * CPT
        pltpu.sync_copy(hcat_hbm.at[pl.ds(hoff + colbase * NP, CPT * NP)], hs)

        def zb(i, _):
            os_[pl.ds(i * 16, 16)] = zero
            return 0

        lax.fori_loop(0, CPT * NP // 16, zb, 0)

        def chunk_b(j, _):
            base = j * CHUNK
            pltpu.sync_copy(src_hbm.at[pl.ds(base, CHUNK)], sb)
            pltpu.sync_copy(dst_hbm.at[pl.ds(base, CHUNK)], db)
            pltpu.sync_copy(alpha_hbm.at[pl.ds(aoff + base, CHUNK)], eb)

            def body_b(i, _):
                s16 = sb[pl.ds(i * 16, 16)]
                d16 = db[pl.ds(i * 16, 16)]
                al = eb[pl.ds(i * 16, 16)]
                for cc in range(CPT):
                    v = plsc.load_gather(hs, [s16 + cc * NP])
                    plsc.addupdate_scatter(os_, [d16 + cc * NP], v * al)
                return 0

            lax.fori_loop(0, CHUNK // 16, body_b, 0)
            return 0

        lax.fori_loop(0, NCB, chunk_b, 0)

        def bias_add(i, _):
            cc = i // (NP // 16)
            b16 = plsc.load_gather(
                biasb, [jnp.full((16,), colbase + cc, jnp.int32)])
            os_[pl.ds(i * 16, 16)] = os_[pl.ds(i * 16, 16)] + b16
            return 0

        lax.fori_loop(0, CPT * NP // 16, bias_add, 0)
        pltpu.sync_copy(
            os_, out_hbm.at[pl.ds(cid * 128 * NP + colbase * NP, CPT * NP)])


_IPW = TP // 32


@functools.partial(
    pl.kernel,
    out_type=jax.ShapeDtypeStruct((4 * TP,), jnp.float32),
    mesh=_mesh,
    compiler_params=_scparams,
    scratch_types=[
        pltpu.VMEM((4 * NP,), jnp.float32),
        pltpu.VMEM((_IPW,), jnp.int32),
        pltpu.VMEM((_IPW,), jnp.int32),
        pltpu.VMEM((_IPW,), jnp.float32),
        pltpu.VMEM((_IPW,), jnp.float32),
        pltpu.VMEM((_IPW,), jnp.float32),
        pltpu.VMEM((_IPW,), jnp.float32),
    ],
)
def _gather_heads(z_hbm, t_hbm, c_hbm, y_hbm, zt, ti, ci, o0, o1, o2, o3):
    cid = lax.axis_index("c")
    sid = lax.axis_index("s")
    w = cid * NT + sid
    pltpu.sync_copy(z_hbm, zt)
    pltpu.sync_copy(t_hbm.at[pl.ds(w * _IPW, _IPW)], ti)
    pltpu.sync_copy(c_hbm.at[pl.ds(w * _IPW, _IPW)], ci)

    def body(i, _):
        t16 = ti[pl.ds(i * 16, 16)]
        c16 = ci[pl.ds(i * 16, 16)]
        o0[pl.ds(i * 16, 16)] = plsc.load_gather(zt, [t16])
        o1[pl.ds(i * 16, 16)] = plsc.load_gather(zt, [t16 + 3 * NP])
        o2[pl.ds(i * 16, 16)] = plsc.load_gather(zt, [c16 + NP])
        o3[pl.ds(i * 16, 16)] = plsc.load_gather(zt, [c16 + 2 * NP])
        return 0

    lax.fori_loop(0, _IPW // 16, body, 0)
    pltpu.sync_copy(o0, y_hbm.at[pl.ds(0 * TP + w * _IPW, _IPW)])
    pltpu.sync_copy(o1, y_hbm.at[pl.ds(1 * TP + w * _IPW, _IPW)])
    pltpu.sync_copy(o2, y_hbm.at[pl.ds(2 * TP + w * _IPW, _IPW)])
    pltpu.sync_copy(o3, y_hbm.at[pl.ds(3 * TP + w * _IPW, _IPW)])


def kernel(x, edge_index, fake_x, treat_idx, control_idx,
           W1, att_src1, att_dst1, b1,
           W2, att_src2, att_dst2, b2,
           Wy1, by1, Wy0, by0):
    loopi = jnp.arange(N, dtype=jnp.int32)
    padi = jnp.full((PADE,), N, jnp.int32)
    srcp = jnp.concatenate([edge_index[0], loopi, padi])
    dstp = jnp.concatenate([edge_index[1], loopi, padi])

    xt = jnp.zeros((2, 128, NP), jnp.float32)
    xt = xt.at[0, :, :N].set(x.T).at[1, :, :N].set(fake_x.T)

    a1m = jnp.stack([att_src1, att_dst1], axis=1)
    a2m = jnp.stack([att_src2, att_dst2], axis=1)

    h1 = _dense(xt, W1, a1m, False)
    out1 = _edge_kernel(h1.reshape(-1), srcp, dstp, b1)[0].reshape(2, 128, NP)
    h2 = _dense(out1, W2, a2m, True)
    out2 = _edge_kernel(h2.reshape(-1), srcp, dstp, b2)[0].reshape(2, 128, NP)

    wy = jnp.concatenate([Wy1, Wy0], axis=1)
    by = jnp.stack([by1[0], by0[0]]).reshape(2, 1)
    z = _heads(out2, wy, by)

    tp = jnp.zeros((TP,), jnp.int32).at[:TSEL].set(treat_idx)
    cp = jnp.zeros((TP,), jnp.int32).at[:TSEL].set(control_idx)
    ys = _gather_heads(z.reshape(-1), tp, cp)

    y1 = ys[0 * TP:0 * TP + TSEL]
    yc0 = ys[1 * TP:1 * TP + TSEL]
    y0 = ys[2 * TP:2 * TP + TSEL]
    yc1 = ys[3 * TP:3 * TP + TSEL]
    xZ2 = out2[0, :, :N].T
    xfZ2 = out2[1, :, :N].T
    return (y1, yc0, y0, yc1, xZ2, xfZ2)
